# baseline (speedup 1.0000x reference)
import functools

import jax
import jax.numpy as jnp
from jax import lax
from jax.experimental import pallas as pl
from jax.experimental.pallas import tpu as pltpu
from jax.experimental.pallas import tpu_sc as plsc

B, T, VOCAB, D, UNITS = 128, 200, 100000, 256, 1024
TB = 8
NW = 32
T1 = 24
T2 = T - T1


def _gather_body(tok_per_w, chunk, idx_hbm, table_hbm, out_hbm,
                 idx_v, rows0, rows1, gsem0, gsem1, ssem0, ssem1):
    n_chunks = tok_per_w // chunk
    wid = lax.axis_index("s") * 2 + lax.axis_index("c")
    base = wid * tok_per_w
    pltpu.sync_copy(idx_hbm.at[pl.ds(base, tok_per_w)], idx_v)
    rows = (rows0, rows1)
    gsem = (gsem0, gsem1)
    ssem = (ssem0, ssem1)
    pending = [None, None]
    for c in range(n_chunks):
        b = c % 2
        if pending[b] is not None:
            pending[b].wait()
        pltpu.async_copy(
            table_hbm.at[idx_v.at[pl.ds(c * chunk, chunk)]], rows[b], gsem[b]
        ).wait()
        pending[b] = pltpu.async_copy(
            rows[b], out_hbm.at[pl.ds(base + c * chunk, chunk)], ssem[b]
        )
    for b in range(2):
        if pending[b] is not None:
            pending[b].wait()


def _sc_gather(idx, table, n_tok, chunk):
    tok_per_w = n_tok // NW
    mesh = plsc.VectorSubcoreMesh(core_axis_name="c", subcore_axis_name="s")
    return pl.kernel(
        functools.partial(_gather_body, tok_per_w, chunk),
        mesh=mesh,
        out_type=jax.ShapeDtypeStruct((n_tok, D), jnp.float32),
        scratch_types=[
            pltpu.VMEM((tok_per_w,), jnp.int32),
            pltpu.VMEM((chunk, D), jnp.float32),
            pltpu.VMEM((chunk, D), jnp.float32),
            pltpu.SemaphoreType.DMA,
            pltpu.SemaphoreType.DMA,
            pltpu.SemaphoreType.DMA,
            pltpu.SemaphoreType.DMA,
        ],
    )(idx, table)


def _gru_body(has_prev, *refs):
    if has_prev:
        x_ref, w_ref, r_ref, b_ref, h0_ref, _seq_in, seq_ref, hout_ref, h_ref = refs
    else:
        x_ref, w_ref, r_ref, b_ref, h0_ref, seq_ref, hout_ref, h_ref = refs
    t = pl.program_id(0)

    @pl.when(t == 0)
    def _():
        h_ref[...] = h0_ref[...]

    b0 = b_ref[0:1, :]
    b1 = b_ref[1:2, :]
    xall = x_ref[...].reshape(TB * B, D)
    xp_all = jnp.dot(xall, w_ref[...], preferred_element_type=jnp.float32) + b0
    h = h_ref[...]
    U2 = 2 * UNITS
    for j in range(TB):
        xp = xp_all[j * B:(j + 1) * B, :]
        rp = jnp.dot(h, r_ref[...], preferred_element_type=jnp.float32) + b1
        z = jax.nn.sigmoid(xp[:, :UNITS] + rp[:, :UNITS])
        r = jax.nn.sigmoid(xp[:, UNITS:U2] + rp[:, UNITS:U2])
        hh = jnp.tanh(xp[:, U2:] + r * rp[:, U2:])
        h = hh + z * (h - hh)
        seq_ref[:, j, :] = h
    h_ref[...] = h
    hout_ref[...] = h


def _gru_stage(x_tbd, w, r, b, h0, seq_prev, t_len, t_off):
    has_prev = seq_prev is not None
    in_specs = [
        pl.BlockSpec((TB, B, D), lambda t: (t, 0, 0)),
        pl.BlockSpec((D, 3 * UNITS), lambda t: (0, 0)),
        pl.BlockSpec((UNITS, 3 * UNITS), lambda t: (0, 0)),
        pl.BlockSpec((2, 3 * UNITS), lambda t: (0, 0)),
        pl.BlockSpec((B, UNITS), lambda t: (0, 0)),
    ]
    args = [x_tbd, w, r, b, h0]
    if has_prev:
        in_specs.append(pl.BlockSpec(memory_space=pl.ANY))
        args.append(seq_prev)
    off = t_off // TB
    return pl.pallas_call(
        functools.partial(_gru_body, has_prev),
        grid=(t_len // TB,),
        in_specs=in_specs,
        out_specs=[
            pl.BlockSpec((B, TB, UNITS), lambda t: (0, t + off, 0)),
            pl.BlockSpec((B, UNITS), lambda t: (0, 0)),
        ],
        out_shape=[
            jax.ShapeDtypeStruct((B, T, UNITS), jnp.float32),
            jax.ShapeDtypeStruct((B, UNITS), jnp.float32),
        ],
        scratch_shapes=[pltpu.VMEM((B, UNITS), jnp.float32)],
        input_output_aliases={5: 0} if has_prev else {},
    )(*args)


def kernel(inputs, training, embedding, kernel, recurrent_kernel, bias):
    idx = inputs.T.reshape(-1).astype(jnp.int32)
    idx1, idx2 = idx[:T1 * B], idx[T1 * B:]
    x1 = _sc_gather(idx1, embedding, T1 * B, 96)
    x2 = _sc_gather(idx2, embedding, T2 * B, 88)
    h0 = jnp.zeros((B, UNITS), dtype=jnp.float32)
    seq1, h1 = _gru_stage(x1.reshape(T1, B, D), kernel, recurrent_kernel,
                          bias, h0, None, T1, 0)
    seq2, h2 = _gru_stage(x2.reshape(T2, B, D), kernel, recurrent_kernel,
                          bias, h1, seq1, T2, T1)
    return seq2, h2


# speedup vs baseline: 1.0492x; 1.0492x over previous
import functools

import jax
import jax.numpy as jnp
from jax import lax
from jax.experimental import pallas as pl
from jax.experimental.pallas import tpu as pltpu
from jax.experimental.pallas import tpu_sc as plsc

B, T, VOCAB, D, UNITS = 128, 200, 100000, 256, 1024
TB = 8
NW = 32
T1 = 24
T2 = T - T1


def _gather_body(tok_per_w, chunk, idx_hbm, table_hbm, out_hbm,
                 idx_v, rows0, rows1, gsem0, gsem1, ssem0, ssem1):
    n_chunks = tok_per_w // chunk
    wid = lax.axis_index("s") * 2 + lax.axis_index("c")
    base = wid * tok_per_w
    pltpu.sync_copy(idx_hbm.at[pl.ds(base, tok_per_w)], idx_v)
    rows = (rows0, rows1)
    gsem = (gsem0, gsem1)
    ssem = (ssem0, ssem1)
    pending = [None, None]
    for c in range(n_chunks):
        b = c % 2
        if pending[b] is not None:
            pending[b].wait()
        pltpu.async_copy(
            table_hbm.at[idx_v.at[pl.ds(c * chunk, chunk)]], rows[b], gsem[b]
        ).wait()
        pending[b] = pltpu.async_copy(
            rows[b], out_hbm.at[pl.ds(base + c * chunk, chunk)], ssem[b]
        )
    for b in range(2):
        if pending[b] is not None:
            pending[b].wait()


def _sc_gather(idx, table, n_tok, chunk):
    tok_per_w = n_tok // NW
    mesh = plsc.VectorSubcoreMesh(core_axis_name="c", subcore_axis_name="s")
    return pl.kernel(
        functools.partial(_gather_body, tok_per_w, chunk),
        mesh=mesh,
        out_type=jax.ShapeDtypeStruct((n_tok, D), jnp.float32),
        scratch_types=[
            pltpu.VMEM((tok_per_w,), jnp.int32),
            pltpu.VMEM((chunk, D), jnp.float32),
            pltpu.VMEM((chunk, D), jnp.float32),
            pltpu.SemaphoreType.DMA,
            pltpu.SemaphoreType.DMA,
            pltpu.SemaphoreType.DMA,
            pltpu.SemaphoreType.DMA,
        ],
    )(idx, table)


def _gru_body(has_prev, *refs):
    if has_prev:
        x_ref, w_ref, r_ref, b_ref, h0_ref, _seq_in, seq_ref, hout_ref, h_ref = refs
    else:
        x_ref, w_ref, r_ref, b_ref, h0_ref, seq_ref, hout_ref, h_ref = refs
    t = pl.program_id(0)

    @pl.when(t == 0)
    def _():
        h_ref[...] = h0_ref[...]

    b0 = b_ref[0:1, :]
    b1 = b_ref[1:2, :]
    xall = x_ref[...].reshape(TB * B, D).astype(jnp.bfloat16)
    xp_all = jnp.dot(xall, w_ref[...], preferred_element_type=jnp.float32) + b0
    h = h_ref[...]
    U2 = 2 * UNITS
    for j in range(TB):
        xp = xp_all[j * B:(j + 1) * B, :]
        rp = jnp.dot(h.astype(jnp.bfloat16), r_ref[...],
                     preferred_element_type=jnp.float32) + b1
        z = jax.nn.sigmoid(xp[:, :UNITS] + rp[:, :UNITS])
        r = jax.nn.sigmoid(xp[:, UNITS:U2] + rp[:, UNITS:U2])
        hh = jnp.tanh(xp[:, U2:] + r * rp[:, U2:])
        h = hh + z * (h - hh)
        seq_ref[:, j, :] = h
    h_ref[...] = h
    hout_ref[...] = h


def _gru_stage(x_tbd, w, r, b, h0, seq_prev, t_len, t_off):
    has_prev = seq_prev is not None
    in_specs = [
        pl.BlockSpec((TB, B, D), lambda t: (t, 0, 0)),
        pl.BlockSpec((D, 3 * UNITS), lambda t: (0, 0)),
        pl.BlockSpec((UNITS, 3 * UNITS), lambda t: (0, 0)),
        pl.BlockSpec((2, 3 * UNITS), lambda t: (0, 0)),
        pl.BlockSpec((B, UNITS), lambda t: (0, 0)),
    ]
    args = [x_tbd, w, r, b, h0]
    if has_prev:
        in_specs.append(pl.BlockSpec(memory_space=pl.ANY))
        args.append(seq_prev)
    off = t_off // TB
    return pl.pallas_call(
        functools.partial(_gru_body, has_prev),
        grid=(t_len // TB,),
        in_specs=in_specs,
        out_specs=[
            pl.BlockSpec((B, TB, UNITS), lambda t: (0, t + off, 0)),
            pl.BlockSpec((B, UNITS), lambda t: (0, 0)),
        ],
        out_shape=[
            jax.ShapeDtypeStruct((B, T, UNITS), jnp.float32),
            jax.ShapeDtypeStruct((B, UNITS), jnp.float32),
        ],
        scratch_shapes=[pltpu.VMEM((B, UNITS), jnp.float32)],
        input_output_aliases={5: 0} if has_prev else {},
    )(*args)


def kernel(inputs, training, embedding, kernel, recurrent_kernel, bias):
    idx = inputs.T.reshape(-1).astype(jnp.int32)
    idx1, idx2 = idx[:T1 * B], idx[T1 * B:]
    x1 = _sc_gather(idx1, embedding, T1 * B, 96)
    x2 = _sc_gather(idx2, embedding, T2 * B, 88)
    h0 = jnp.zeros((B, UNITS), dtype=jnp.float32)
    wb = kernel.astype(jnp.bfloat16)
    rb = recurrent_kernel.astype(jnp.bfloat16)
    seq1, h1 = _gru_stage(x1.reshape(T1, B, D), wb, rb, bias, h0, None, T1, 0)
    seq2, h2 = _gru_stage(x2.reshape(T2, B, D), wb, rb, bias, h1, seq1, T2, T1)
    return seq2, h2


# speedup vs baseline: 1.0525x; 1.0031x over previous
import functools

import jax
import jax.numpy as jnp
from jax import lax
from jax.experimental import pallas as pl
from jax.experimental.pallas import tpu as pltpu
from jax.experimental.pallas import tpu_sc as plsc

B, T, VOCAB, D, UNITS = 128, 200, 100000, 256, 1024
TB = 8
NW = 32
T1 = 24
T2 = T - T1


def _gather_body(tok_per_w, chunk, idx_hbm, table_hbm, out_hbm,
                 idx_v, rows0, rows1, gsem0, gsem1, ssem0, ssem1):
    n_chunks = tok_per_w // chunk
    wid = lax.axis_index("s") * 2 + lax.axis_index("c")
    base = wid * tok_per_w
    pltpu.sync_copy(idx_hbm.at[pl.ds(base, tok_per_w)], idx_v)
    rows = (rows0, rows1)
    gsem = (gsem0, gsem1)
    ssem = (ssem0, ssem1)
    pending = [None, None]
    for c in range(n_chunks):
        b = c % 2
        if pending[b] is not None:
            pending[b].wait()
        pltpu.async_copy(
            table_hbm.at[idx_v.at[pl.ds(c * chunk, chunk)]], rows[b], gsem[b]
        ).wait()
        pending[b] = pltpu.async_copy(
            rows[b], out_hbm.at[pl.ds(base + c * chunk, chunk)], ssem[b]
        )
    for b in range(2):
        if pending[b] is not None:
            pending[b].wait()


def _sc_gather(idx, table, n_tok, chunk):
    tok_per_w = n_tok // NW
    mesh = plsc.VectorSubcoreMesh(core_axis_name="c", subcore_axis_name="s")
    return pl.kernel(
        functools.partial(_gather_body, tok_per_w, chunk),
        mesh=mesh,
        out_type=jax.ShapeDtypeStruct((n_tok, D), jnp.float32),
        scratch_types=[
            pltpu.VMEM((tok_per_w,), jnp.int32),
            pltpu.VMEM((chunk, D), jnp.float32),
            pltpu.VMEM((chunk, D), jnp.float32),
            pltpu.SemaphoreType.DMA,
            pltpu.SemaphoreType.DMA,
            pltpu.SemaphoreType.DMA,
            pltpu.SemaphoreType.DMA,
        ],
    )(idx, table)


def _gru_body(has_prev, *refs):
    if has_prev:
        x_ref, w_ref, r_ref, b_ref, h0_ref, _seq_in, seq_ref, hout_ref, h_ref = refs
    else:
        x_ref, w_ref, r_ref, b_ref, h0_ref, seq_ref, hout_ref, h_ref = refs
    t = pl.program_id(0)

    @pl.when(t == 0)
    def _():
        h_ref[...] = h0_ref[...]

    b0 = b_ref[0:1, :]
    b1 = b_ref[1:2, :]
    xall = x_ref[...].reshape(TB * B, D).astype(jnp.bfloat16)
    xp_all = jnp.dot(xall, w_ref[...], preferred_element_type=jnp.float32) + b0
    h = h_ref[...]
    U2 = 2 * UNITS
    for j in range(TB):
        xp = xp_all[j * B:(j + 1) * B, :]
        rp = jnp.dot(h.astype(jnp.bfloat16), r_ref[...],
                     preferred_element_type=jnp.float32) + b1
        z = jax.nn.sigmoid(xp[:, :UNITS] + rp[:, :UNITS])
        r = jax.nn.sigmoid(xp[:, UNITS:U2] + rp[:, UNITS:U2])
        hh = jnp.tanh(xp[:, U2:] + r * rp[:, U2:])
        h = hh + z * (h - hh)
        seq_ref[:, j, :] = h
    h_ref[...] = h
    hout_ref[...] = h


def _gru_stage(x_tbd, w, r, b, h0, seq_prev, t_len, t_off):
    has_prev = seq_prev is not None
    in_specs = [
        pl.BlockSpec((TB, B, D), lambda t: (t, 0, 0)),
        pl.BlockSpec((D, 3 * UNITS), lambda t: (0, 0)),
        pl.BlockSpec((UNITS, 3 * UNITS), lambda t: (0, 0)),
        pl.BlockSpec((2, 3 * UNITS), lambda t: (0, 0)),
        pl.BlockSpec((B, UNITS), lambda t: (0, 0)),
    ]
    args = [x_tbd, w, r, b, h0]
    if has_prev:
        in_specs.append(pl.BlockSpec(memory_space=pl.ANY))
        args.append(seq_prev)
    off = t_off // TB
    return pl.pallas_call(
        functools.partial(_gru_body, has_prev),
        grid=(t_len // TB,),
        in_specs=in_specs,
        out_specs=[
            pl.BlockSpec((B, TB, UNITS), lambda t: (0, t + off, 0)),
            pl.BlockSpec((B, UNITS), lambda t: (0, 0)),
        ],
        out_shape=[
            jax.ShapeDtypeStruct((B, T, UNITS), jnp.float32),
            jax.ShapeDtypeStruct((B, UNITS), jnp.float32),
        ],
        scratch_shapes=[pltpu.VMEM((B, UNITS), jnp.float32)],
        input_output_aliases={5: 0} if has_prev else {},
    )(*args)


def kernel(inputs, training, embedding, kernel, recurrent_kernel, bias):
    wb = kernel.astype(jnp.bfloat16)
    rb = recurrent_kernel.astype(jnp.bfloat16)
    idx = inputs.T.reshape(-1).astype(jnp.int32)
    idx1, idx2 = idx[:T1 * B], idx[T1 * B:]
    x1 = _sc_gather(idx1, embedding, T1 * B, 96)
    x2 = _sc_gather(idx2, embedding, T2 * B, 88)
    h0 = jnp.zeros((B, UNITS), dtype=jnp.float32)
    seq1, h1 = _gru_stage(x1.reshape(T1, B, D), wb, rb, bias, h0, None, T1, 0)
    seq2, h2 = _gru_stage(x2.reshape(T2, B, D), wb, rb, bias, h1, seq1, T2, T1)
    return seq2, h2
